# baseline (speedup 1.0000x reference)
import dataclasses
import functools

import jax
import jax.numpy as jnp
from jax import lax
from jax.experimental import pallas as pl
from jax.experimental.pallas import tpu as pltpu
from jax.experimental.pallas import tpu_sc as plsc

N = 10000
NP = 10240
D = 128
H = 8
DE = D + H
E = 320000
ES = E + N
NCORE = 2
NSUB = 16
NWORK = NCORE * NSUB
C = 96
K = 108
EP = NWORK * C * K
RPT = NP // NSUB

_PREC = lax.Precision.HIGHEST


_BLK = 1024
_NBLK = NP // _BLK


def _tc1_body(x_ref, w_ref, asm_ref, adm_ref, h_ref, s_ref, d_ref):
    h = jnp.dot(x_ref[...], w_ref[...], preferred_element_type=jnp.float32,
                precision=_PREC)
    h_ref[...] = h
    s_ref[...] = jnp.dot(h, asm_ref[...], preferred_element_type=jnp.float32,
                         precision=_PREC)
    d_ref[...] = jnp.dot(h, adm_ref[...], preferred_element_type=jnp.float32,
                         precision=_PREC)


def _tc1(x_pad, w1, a_src, a_dst):
    return pl.pallas_call(
        _tc1_body,
        grid=(_NBLK,),
        in_specs=[
            pl.BlockSpec((_BLK, D), lambda i: (i, 0)),
            pl.BlockSpec((D, D), lambda i: (0, 0)),
            pl.BlockSpec((D, H), lambda i: (0, 0)),
            pl.BlockSpec((D, H), lambda i: (0, 0)),
        ],
        out_specs=[
            pl.BlockSpec((_BLK, D), lambda i: (i, 0)),
            pl.BlockSpec((_BLK, H), lambda i: (i, 0)),
            pl.BlockSpec((_BLK, H), lambda i: (i, 0)),
        ],
        out_shape=[
            jax.ShapeDtypeStruct((NP, D), jnp.float32),
            jax.ShapeDtypeStruct((NP, H), jnp.float32),
            jax.ShapeDtypeStruct((NP, H), jnp.float32),
        ],
    )(x_pad, w1, a_src, a_dst)


def _norm_ln_elu(acc, den_exp, bias, g, b, x_res):
    o = acc / (den_exp + 1e-16) + bias
    mu = jnp.mean(o, axis=-1, keepdims=True)
    var = jnp.mean((o - mu) ** 2, axis=-1, keepdims=True)
    y = (o - mu) * lax.rsqrt(var + 1e-5) * g + b
    y = jnp.where(y > 0, y, jnp.exp(y) - 1.0)
    return x_res + y


def _tc2_body(ap_ref, x_ref, bias_ref, g_ref, b_ref, exp_ref,
              w2_ref, asm_ref, adm_ref, x1_ref, h2_ref, s_ref, d_ref):
    acc = ap_ref[0, :, :D] + ap_ref[1, :, :D]
    den = ap_ref[0, :, D:] + ap_ref[1, :, D:]
    den_exp = jnp.dot(den, exp_ref[...], preferred_element_type=jnp.float32,
                      precision=_PREC)
    x1 = _norm_ln_elu(acc, den_exp, bias_ref[...], g_ref[...], b_ref[...],
                      x_ref[...])
    x1_ref[...] = x1
    h2 = jnp.dot(x1, w2_ref[...], preferred_element_type=jnp.float32,
                 precision=_PREC)
    h2_ref[...] = h2
    s_ref[...] = jnp.dot(h2, asm_ref[...], preferred_element_type=jnp.float32,
                         precision=_PREC)
    d_ref[...] = jnp.dot(h2, adm_ref[...], preferred_element_type=jnp.float32,
                         precision=_PREC)


def _tc2(acc_p, x_pad, bias1, g1, b1, expand1, w2, a_src2, a_dst2):
    return pl.pallas_call(
        _tc2_body,
        grid=(_NBLK,),
        in_specs=[
            pl.BlockSpec((NCORE, _BLK, DE), lambda i: (0, i, 0)),
            pl.BlockSpec((_BLK, D), lambda i: (i, 0)),
            pl.BlockSpec((D,), lambda i: (0,)),
            pl.BlockSpec((D,), lambda i: (0,)),
            pl.BlockSpec((D,), lambda i: (0,)),
            pl.BlockSpec((H, D), lambda i: (0, 0)),
            pl.BlockSpec((D, D), lambda i: (0, 0)),
            pl.BlockSpec((D, H), lambda i: (0, 0)),
            pl.BlockSpec((D, H), lambda i: (0, 0)),
        ],
        out_specs=[
            pl.BlockSpec((_BLK, D), lambda i: (i, 0)),
            pl.BlockSpec((_BLK, D), lambda i: (i, 0)),
            pl.BlockSpec((_BLK, H), lambda i: (i, 0)),
            pl.BlockSpec((_BLK, H), lambda i: (i, 0)),
        ],
        out_shape=[
            jax.ShapeDtypeStruct((NP, D), jnp.float32),
            jax.ShapeDtypeStruct((NP, D), jnp.float32),
            jax.ShapeDtypeStruct((NP, H), jnp.float32),
            jax.ShapeDtypeStruct((NP, H), jnp.float32),
        ],
    )(acc_p, x_pad, bias1, g1, b1, expand1, w2, a_src2, a_dst2)


def _tc3_body(ap_ref, x1_ref, bias_ref, g_ref, b_ref, exp_ref,
              out_ref):
    acc = ap_ref[0, :, :D] + ap_ref[1, :, :D]
    den = ap_ref[0, :, D:] + ap_ref[1, :, D:]
    den_exp = jnp.dot(den, exp_ref[...], preferred_element_type=jnp.float32,
                      precision=_PREC)
    out_ref[...] = _norm_ln_elu(acc, den_exp, bias_ref[...], g_ref[...],
                                b_ref[...], x1_ref[...])


def _tc3(acc_p, x1, bias2, g2, b2, expand2):
    return pl.pallas_call(
        _tc3_body,
        grid=(_NBLK,),
        in_specs=[
            pl.BlockSpec((NCORE, _BLK, DE), lambda i: (0, i, 0)),
            pl.BlockSpec((_BLK, D), lambda i: (i, 0)),
            pl.BlockSpec((D,), lambda i: (0,)),
            pl.BlockSpec((D,), lambda i: (0,)),
            pl.BlockSpec((D,), lambda i: (0,)),
            pl.BlockSpec((H, D), lambda i: (0, 0)),
        ],
        out_specs=pl.BlockSpec((_BLK, D), lambda i: (i, 0)),
        out_shape=jax.ShapeDtypeStruct((NP, D), jnp.float32),
    )(acc_p, x1, bias2, g2, b2, expand2)


def _sc_compiler_params():
    cp = pltpu.CompilerParams()
    fields = pltpu.CompilerParams.__dataclass_fields__
    if "needs_layout_passes" in fields:
        cp = dataclasses.replace(cp, needs_layout_passes=False)
    if "use_tc_tiling_on_sc" in fields:
        cp = dataclasses.replace(cp, use_tc_tiling_on_sc=False)
    return cp


NBUF = 3
assert K % NBUF == 0
_TSTEPS = K // NBUF


def _make_sc_agg(head_map):
    mesh = plsc.VectorSubcoreMesh(core_axis_name="c", subcore_axis_name="s")

    @functools.partial(
        pl.kernel,
        compiler_params=_sc_compiler_params(),
        out_type=jax.ShapeDtypeStruct((NCORE, NP, DE), jnp.float32),
        mesh=mesh,
        scratch_types=[
            pltpu.VMEM((NBUF, 2, C), jnp.int32),
            pltpu.VMEM((NBUF, C, H), jnp.float32),
            pltpu.VMEM((NBUF, C, DE), jnp.float32),
            pltpu.VMEM_SHARED((NP, DE), jnp.float32),
            pltpu.SemaphoreType.DMA((NBUF,)),
            pltpu.SemaphoreType.DMA((NBUF,)),
            pltpu.SemaphoreType.DMA((NBUF,)),
        ],
    )
    def sc_agg(ei_h, adst_h, htab_h, z_h,
               acc_o,
               idx_v, ad_v, hr_v, acc_sh,
               sem_ad, sem_hr, sem_sa):
        cid = lax.axis_index("c")
        sid = lax.axis_index("s")
        wid = sid * NCORE + cid
        r0 = sid * RPT

        pltpu.sync_copy(z_h.at[pl.ds(r0, RPT)], acc_sh.at[pl.ds(r0, RPT)])

        def prefetch(j, b):
            pltpu.sync_copy(ei_h.at[wid * K + j], idx_v.at[b])
            pltpu.async_copy(adst_h.at[idx_v.at[b, 1]], ad_v.at[b],
                             sem_ad.at[b])
            pltpu.async_copy(htab_h.at[idx_v.at[b, 0]], hr_v.at[b],
                             sem_hr.at[b])

        def wait_gathers(b):
            pltpu.make_async_copy(adst_h.at[idx_v.at[b, 1]], ad_v.at[b],
                                  sem_ad.at[b]).wait()
            pltpu.make_async_copy(htab_h.at[idx_v.at[b, 0]], hr_v.at[b],
                                  sem_hr.at[b]).wait()

        def start_scatters(b):
            pltpu.async_copy(hr_v.at[b], acc_sh.at[idx_v.at[b, 1]],
                             sem_sa.at[b], add=True)

        def wait_scatters(b):
            pltpu.make_async_copy(hr_v.at[b], acc_sh.at[idx_v.at[b, 1]],
                                  sem_sa.at[b]).wait()

        prefetch(0, 0)
        prefetch(1, 1)
        plsc.subcore_barrier()

        lanes = lax.iota(jnp.int32, 16)
        rowpat = lanes // H
        colpat = lanes % H
        zeros16 = lanes * 0
        need_lanes = sorted({hm + 8 * le for hm in head_map for le in (0, 1)})
        lane_consts = {l: zeros16 + l for l in need_lanes}

        def process(b):
            wait_gathers(b)

            hrb = hr_v.at[b]
            adb = ad_v.at[b]

            @pl.loop(0, C // 2)
            def _pair(p):
                rows = rowpat + p * 2
                a = (plsc.load_gather(hrb, [rows, colpat + D])
                     + plsc.load_gather(adb, [rows, colpat]))
                a = jnp.where(a > 0, a, a * 0.2)
                e = jnp.exp(a)
                plsc.store_scatter(hrb, [rows, colpat + D], e)
                for le in range(2):
                    row = hr_v.at[b].at[p * 2 + le]
                    for g in range(H):
                        lane = lane_consts[head_map[g] + 8 * le]
                        vg = e.at[lane].get(mode="promise_in_bounds")
                        sl = pl.ds(g * 16, 16)
                        row[sl] = row[sl] * vg

            start_scatters(b)

        @pl.loop(0, _TSTEPS)
        def _step(t):
            j0 = t * NBUF
            for u in range(NBUF):
                j = j0 + u
                b = u
                process(b)
                bn = (u + 2) % NBUF
                if u == 0:
                    @pl.when(t >= 1)
                    def _():
                        wait_scatters(bn)
                    prefetch(j + 2, bn)
                else:
                    @pl.when(t < _TSTEPS - 1)
                    def _():
                        wait_scatters(bn)
                        prefetch(j + 2, bn)

        for u in range(NBUF):
            wait_scatters(u)
        plsc.subcore_barrier()
        pltpu.sync_copy(acc_sh.at[pl.ds(r0, RPT)],
                        acc_o.at[cid, pl.ds(r0, RPT)])

    return sc_agg


_sc_agg_l1 = _make_sc_agg(tuple(range(H)))
_sc_agg_l2 = _make_sc_agg((0,) * H)


def _block_diag_att(att):
    heads, hid = att.shape
    return (att[:, :, None] * jnp.eye(heads, dtype=att.dtype)[:, None, :]
            ).reshape(heads * hid, heads)


def kernel(x, edge_index, W1, att_src1, att_dst1, bias1, g1, b1,
           W2, att_src2, att_dst2, bias2, g2, b2):
    f32 = jnp.float32
    x_pad = jnp.zeros((NP, D), f32).at[:N].set(x)
    loop = jnp.arange(N, dtype=jnp.int32)
    padv = jnp.full((EP - ES,), N, jnp.int32)
    src = jnp.concatenate([edge_index[0], loop, padv]).reshape(NWORK * K, 1, C)
    dst = jnp.concatenate([edge_index[1], loop, padv]).reshape(NWORK * K, 1, C)
    ei = jnp.concatenate([src, dst], axis=1)

    a_src1 = _block_diag_att(att_src1)
    a_dst1 = _block_diag_att(att_dst1)
    zcol = jnp.zeros((D, H - 1), f32)
    a_src2 = jnp.concatenate([att_src2.reshape(D, 1), zcol], axis=1)
    a_dst2 = jnp.concatenate([att_dst2.reshape(D, 1), zcol], axis=1)
    expand1 = jnp.kron(jnp.eye(H, dtype=f32), jnp.ones((1, 16), f32))
    expand2 = jnp.concatenate(
        [jnp.ones((1, D), f32), jnp.zeros((H - 1, D), f32)], axis=0)

    z = jnp.zeros((NP, DE), f32)

    h1, asrc1, adst1 = _tc1(x_pad, W1, a_src1, a_dst1)
    htab1 = jnp.concatenate([h1, asrc1], axis=1)
    acc1 = _sc_agg_l1(ei, adst1, htab1, z)
    x1, h2, asrc2, adst2 = _tc2(acc1, x_pad, bias1, g1, b1, expand1,
                                W2, a_src2, a_dst2)
    htab2 = jnp.concatenate([h2, asrc2], axis=1)
    acc2 = _sc_agg_l2(ei, adst2, htab2, z)
    out_pad = _tc3(acc2, x1, bias2, g2, b2, expand2)
    return out_pad[:N]


# speedup vs baseline: 1.0961x; 1.0961x over previous
import dataclasses
import functools

import jax
import jax.numpy as jnp
from jax import lax
from jax.experimental import pallas as pl
from jax.experimental.pallas import tpu as pltpu
from jax.experimental.pallas import tpu_sc as plsc

N = 10000
NP = 10240
D = 128
H = 8
DE = D + H
E = 320000
ES = E + N
NCORE = 2
NSUB = 16
NWORK = NCORE * NSUB
C = 80
K = 129
EP = NWORK * C * K
RPT = NP // NSUB

_PREC = lax.Precision.HIGHEST


_BLK = 1024
_NBLK = NP // _BLK


def _tc1_body(x_ref, w_ref, asm_ref, adm_ref, h_ref, s_ref, d_ref):
    h = jnp.dot(x_ref[...], w_ref[...], preferred_element_type=jnp.float32,
                precision=_PREC)
    h_ref[...] = h
    s_ref[...] = jnp.dot(h, asm_ref[...], preferred_element_type=jnp.float32,
                         precision=_PREC)
    d_ref[...] = jnp.dot(h, adm_ref[...], preferred_element_type=jnp.float32,
                         precision=_PREC)


def _tc1(x_pad, w1, a_src, a_dst):
    return pl.pallas_call(
        _tc1_body,
        grid=(_NBLK,),
        in_specs=[
            pl.BlockSpec((_BLK, D), lambda i: (i, 0)),
            pl.BlockSpec((D, D), lambda i: (0, 0)),
            pl.BlockSpec((D, H), lambda i: (0, 0)),
            pl.BlockSpec((D, H), lambda i: (0, 0)),
        ],
        out_specs=[
            pl.BlockSpec((_BLK, D), lambda i: (i, 0)),
            pl.BlockSpec((_BLK, H), lambda i: (i, 0)),
            pl.BlockSpec((_BLK, H), lambda i: (i, 0)),
        ],
        out_shape=[
            jax.ShapeDtypeStruct((NP, D), jnp.float32),
            jax.ShapeDtypeStruct((NP, H), jnp.float32),
            jax.ShapeDtypeStruct((NP, H), jnp.float32),
        ],
    )(x_pad, w1, a_src, a_dst)


def _norm_ln_elu(acc, den_exp, bias, g, b, x_res):
    o = acc / (den_exp + 1e-16) + bias
    mu = jnp.mean(o, axis=-1, keepdims=True)
    var = jnp.mean((o - mu) ** 2, axis=-1, keepdims=True)
    y = (o - mu) * lax.rsqrt(var + 1e-5) * g + b
    y = jnp.where(y > 0, y, jnp.exp(y) - 1.0)
    return x_res + y


def _tc2_body(ap_ref, x_ref, bias_ref, g_ref, b_ref, exp_ref,
              w2_ref, asm_ref, adm_ref, x1_ref, h2_ref, s_ref, d_ref):
    acc = ap_ref[0, :, :D] + ap_ref[1, :, :D]
    den = ap_ref[0, :, D:] + ap_ref[1, :, D:]
    den_exp = jnp.dot(den, exp_ref[...], preferred_element_type=jnp.float32,
                      precision=_PREC)
    x1 = _norm_ln_elu(acc, den_exp, bias_ref[...], g_ref[...], b_ref[...],
                      x_ref[...])
    x1_ref[...] = x1
    h2 = jnp.dot(x1, w2_ref[...], preferred_element_type=jnp.float32,
                 precision=_PREC)
    h2_ref[...] = h2
    s_ref[...] = jnp.dot(h2, asm_ref[...], preferred_element_type=jnp.float32,
                         precision=_PREC)
    d_ref[...] = jnp.dot(h2, adm_ref[...], preferred_element_type=jnp.float32,
                         precision=_PREC)


def _tc2(acc_p, x_pad, bias1, g1, b1, expand1, w2, a_src2, a_dst2):
    return pl.pallas_call(
        _tc2_body,
        grid=(_NBLK,),
        in_specs=[
            pl.BlockSpec((NCORE, _BLK, DE), lambda i: (0, i, 0)),
            pl.BlockSpec((_BLK, D), lambda i: (i, 0)),
            pl.BlockSpec((D,), lambda i: (0,)),
            pl.BlockSpec((D,), lambda i: (0,)),
            pl.BlockSpec((D,), lambda i: (0,)),
            pl.BlockSpec((H, D), lambda i: (0, 0)),
            pl.BlockSpec((D, D), lambda i: (0, 0)),
            pl.BlockSpec((D, H), lambda i: (0, 0)),
            pl.BlockSpec((D, H), lambda i: (0, 0)),
        ],
        out_specs=[
            pl.BlockSpec((_BLK, D), lambda i: (i, 0)),
            pl.BlockSpec((_BLK, D), lambda i: (i, 0)),
            pl.BlockSpec((_BLK, H), lambda i: (i, 0)),
            pl.BlockSpec((_BLK, H), lambda i: (i, 0)),
        ],
        out_shape=[
            jax.ShapeDtypeStruct((NP, D), jnp.float32),
            jax.ShapeDtypeStruct((NP, D), jnp.float32),
            jax.ShapeDtypeStruct((NP, H), jnp.float32),
            jax.ShapeDtypeStruct((NP, H), jnp.float32),
        ],
    )(acc_p, x_pad, bias1, g1, b1, expand1, w2, a_src2, a_dst2)


def _tc3_body(ap_ref, x1_ref, bias_ref, g_ref, b_ref, exp_ref,
              out_ref):
    acc = ap_ref[0, :, :D] + ap_ref[1, :, :D]
    den = ap_ref[0, :, D:] + ap_ref[1, :, D:]
    den_exp = jnp.dot(den, exp_ref[...], preferred_element_type=jnp.float32,
                      precision=_PREC)
    out_ref[...] = _norm_ln_elu(acc, den_exp, bias_ref[...], g_ref[...],
                                b_ref[...], x1_ref[...])


def _tc3(acc_p, x1, bias2, g2, b2, expand2):
    return pl.pallas_call(
        _tc3_body,
        grid=(_NBLK,),
        in_specs=[
            pl.BlockSpec((NCORE, _BLK, DE), lambda i: (0, i, 0)),
            pl.BlockSpec((_BLK, D), lambda i: (i, 0)),
            pl.BlockSpec((D,), lambda i: (0,)),
            pl.BlockSpec((D,), lambda i: (0,)),
            pl.BlockSpec((D,), lambda i: (0,)),
            pl.BlockSpec((H, D), lambda i: (0, 0)),
        ],
        out_specs=pl.BlockSpec((_BLK, D), lambda i: (i, 0)),
        out_shape=jax.ShapeDtypeStruct((NP, D), jnp.float32),
    )(acc_p, x1, bias2, g2, b2, expand2)


def _sc_compiler_params():
    cp = pltpu.CompilerParams()
    fields = pltpu.CompilerParams.__dataclass_fields__
    if "needs_layout_passes" in fields:
        cp = dataclasses.replace(cp, needs_layout_passes=False)
    if "use_tc_tiling_on_sc" in fields:
        cp = dataclasses.replace(cp, use_tc_tiling_on_sc=False)
    return cp


NBUF = 3
assert K % NBUF == 0
_TSTEPS = K // NBUF


def _make_sc_agg(head_map):
    mesh = plsc.VectorSubcoreMesh(core_axis_name="c", subcore_axis_name="s")

    @functools.partial(
        pl.kernel,
        compiler_params=_sc_compiler_params(),
        out_type=jax.ShapeDtypeStruct((NCORE, NP, DE), jnp.float32),
        mesh=mesh,
        scratch_types=[
            pltpu.VMEM((NBUF, 2, C), jnp.int32),
            pltpu.VMEM((NBUF, C, H), jnp.float32),
            pltpu.VMEM((NBUF, C, DE), jnp.float32),
            pltpu.VMEM_SHARED((NP, DE), jnp.float32),
            pltpu.SemaphoreType.DMA((NBUF,)),
            pltpu.SemaphoreType.DMA((NBUF,)),
            pltpu.SemaphoreType.DMA((NBUF,)),
        ],
    )
    def sc_agg(ei_h, adst_h, htab_h, z_h,
               acc_o,
               idx_v, ad_v, hr_v, acc_sh,
               sem_ad, sem_hr, sem_sa):
        cid = lax.axis_index("c")
        sid = lax.axis_index("s")
        wid = sid * NCORE + cid
        r0 = sid * RPT

        pltpu.sync_copy(z_h.at[pl.ds(r0, RPT)], acc_sh.at[pl.ds(r0, RPT)])

        def prefetch(j, b):
            pltpu.sync_copy(ei_h.at[wid * K + j], idx_v.at[b])
            pltpu.async_copy(adst_h.at[idx_v.at[b, 1]], ad_v.at[b],
                             sem_ad.at[b])
            pltpu.async_copy(htab_h.at[idx_v.at[b, 0]], hr_v.at[b],
                             sem_hr.at[b])

        def wait_gathers(b):
            pltpu.make_async_copy(adst_h.at[idx_v.at[b, 1]], ad_v.at[b],
                                  sem_ad.at[b]).wait()
            pltpu.make_async_copy(htab_h.at[idx_v.at[b, 0]], hr_v.at[b],
                                  sem_hr.at[b]).wait()

        def start_scatters(b):
            pltpu.async_copy(hr_v.at[b], acc_sh.at[idx_v.at[b, 1]],
                             sem_sa.at[b], add=True)

        def wait_scatters(b):
            pltpu.make_async_copy(hr_v.at[b], acc_sh.at[idx_v.at[b, 1]],
                                  sem_sa.at[b]).wait()

        prefetch(0, 0)
        prefetch(1, 1)
        plsc.subcore_barrier()

        lanes = lax.iota(jnp.int32, 16)
        rowpat = lanes // H
        colpat = lanes % H
        zeros16 = lanes * 0
        need_lanes = sorted({hm + 8 * le for hm in head_map for le in (0, 1)})
        lane_consts = {l: zeros16 + l for l in need_lanes}

        def process(b):
            wait_gathers(b)

            hrb = hr_v.at[b]
            adb = ad_v.at[b]

            @pl.loop(0, C // 2)
            def _pair(p):
                rows = rowpat + p * 2
                a = (plsc.load_gather(hrb, [rows, colpat + D])
                     + plsc.load_gather(adb, [rows, colpat]))
                a = jnp.where(a > 0, a, a * 0.2)
                e = jnp.exp(a)
                plsc.store_scatter(hrb, [rows, colpat + D], e)
                for le in range(2):
                    row = hr_v.at[b].at[p * 2 + le]
                    for g in range(H):
                        lane = lane_consts[head_map[g] + 8 * le]
                        vg = e.at[lane].get(mode="promise_in_bounds")
                        sl = pl.ds(g * 16, 16)
                        row[sl] = row[sl] * vg

            start_scatters(b)

        @pl.loop(0, _TSTEPS)
        def _step(t):
            j0 = t * NBUF
            for u in range(NBUF):
                j = j0 + u
                b = u
                process(b)
                bn = (u + 2) % NBUF
                if u == 0:
                    @pl.when(t >= 1)
                    def _():
                        wait_scatters(bn)
                    prefetch(j + 2, bn)
                else:
                    @pl.when(t < _TSTEPS - 1)
                    def _():
                        wait_scatters(bn)
                        prefetch(j + 2, bn)

        for u in range(NBUF):
            wait_scatters(u)
        plsc.subcore_barrier()
        pltpu.sync_copy(acc_sh.at[pl.ds(r0, RPT)],
                        acc_o.at[cid, pl.ds(r0, RPT)])

    return sc_agg


_sc_agg_l1 = _make_sc_agg(tuple(range(H)))
_sc_agg_l2 = _make_sc_agg((0,) * H)


def _block_diag_att(att):
    heads, hid = att.shape
    return (att[:, :, None] * jnp.eye(heads, dtype=att.dtype)[:, None, :]
            ).reshape(heads * hid, heads)


def kernel(x, edge_index, W1, att_src1, att_dst1, bias1, g1, b1,
           W2, att_src2, att_dst2, bias2, g2, b2):
    f32 = jnp.float32
    x_pad = jnp.zeros((NP, D), f32).at[:N].set(x)
    loop = jnp.arange(N, dtype=jnp.int32)
    padv = jnp.full((EP - ES,), N, jnp.int32)
    src = jnp.concatenate([edge_index[0], loop, padv]).reshape(NWORK * K, 1, C)
    dst = jnp.concatenate([edge_index[1], loop, padv]).reshape(NWORK * K, 1, C)
    ei = jnp.concatenate([src, dst], axis=1)

    a_src1 = _block_diag_att(att_src1)
    a_dst1 = _block_diag_att(att_dst1)
    zcol = jnp.zeros((D, H - 1), f32)
    a_src2 = jnp.concatenate([att_src2.reshape(D, 1), zcol], axis=1)
    a_dst2 = jnp.concatenate([att_dst2.reshape(D, 1), zcol], axis=1)
    expand1 = jnp.kron(jnp.eye(H, dtype=f32), jnp.ones((1, 16), f32))
    expand2 = jnp.concatenate(
        [jnp.ones((1, D), f32), jnp.zeros((H - 1, D), f32)], axis=0)

    z = jnp.zeros((NP, DE), f32)

    h1, asrc1, adst1 = _tc1(x_pad, W1, a_src1, a_dst1)
    htab1 = jnp.concatenate([h1, asrc1], axis=1)
    acc1 = _sc_agg_l1(ei, adst1, htab1, z)
    x1, h2, asrc2, adst2 = _tc2(acc1, x_pad, bias1, g1, b1, expand1,
                                W2, a_src2, a_dst2)
    htab2 = jnp.concatenate([h2, asrc2], axis=1)
    acc2 = _sc_agg_l2(ei, adst2, htab2, z)
    out_pad = _tc3(acc2, x1, bias2, g2, b2, expand2)
    return out_pad[:N]
